# Initial kernel scaffold; baseline (speedup 1.0000x reference)
import jax
import jax.numpy as jnp
from jax.experimental import pallas as pl


def kernel(x):
    raise NotImplementedError("write your pallas kernel here")


# speedup vs baseline: 7.7111x; 7.7111x over previous
import functools
import math

import jax
import jax.numpy as jnp
from jax import lax
from jax.experimental import pallas as pl
from jax.experimental.pallas import tpu as pltpu
from jax.experimental.pallas import tpu_sc as plsc

B = 128
N = 32768
K = math.ceil(0.05 * N)

NC = 2
NS = 16
L = 16
NW = NC * NS
ROWS_PER_W = B // NW
NV = N // L
NBKT = 256


def _scal(v):
    if getattr(v, "ndim", 0) == 1:
        return jnp.max(v)
    return v


_mesh = plsc.VectorSubcoreMesh(core_axis_name="c", subcore_axis_name="s")


@functools.partial(
    pl.kernel,
    mesh=_mesh,
    out_type=jax.ShapeDtypeStruct((B, N), jnp.float32),
    compiler_params=pltpu.CompilerParams(needs_layout_passes=False),
    scratch_types=[
        pltpu.VMEM((N,), jnp.float32),
        pltpu.VMEM((N,), jnp.int32),
        pltpu.VMEM((N,), jnp.int32),
        pltpu.VMEM((L * NBKT,), jnp.int32),
        pltpu.VMEM((NBKT,), jnp.int32),
    ],
)
def _topk_mask(x_hbm, out_hbm, xv, sa, cb, hist, merged):
    wid = lax.axis_index("s") * NC + lax.axis_index("c")
    lanes = lax.iota(jnp.int32, L)
    laneoff = lanes * NBKT
    ones = jnp.ones((L,), jnp.int32)
    zero16 = jnp.zeros((L,), jnp.int32)
    fzero16 = jnp.zeros((L,), jnp.float32)
    true16 = lanes >= 0

    def clear_hist():
        def body(i, _):
            hist[pl.ds(i * L, L)] = zero16
            return 0
        lax.fori_loop(0, (L * NBKT) // L, body, 0)

    def search(r0):
        def ga_body(t, carry):
            r, found, grp, rg = carry
            g = 15 - t
            def ml(l, acc):
                return acc + hist[pl.ds(l * NBKT + g * L, L)]
            acc = lax.fori_loop(0, L, ml, zero16)
            merged[pl.ds(g * L, L)] = acc
            tot = jnp.sum(acc)
            hit = jnp.logical_and(found == 0, tot >= r)
            grp = jnp.where(hit, g, grp)
            rg = jnp.where(hit, r, rg)
            r = jnp.where(jnp.logical_and(found == 0, jnp.logical_not(hit)),
                          r - tot, r)
            found = jnp.where(hit, jnp.int32(1), found)
            return (r, found, grp, rg)

        r, _found, grp, rg = lax.fori_loop(
            0, 16, ga_body,
            (r0, jnp.int32(0), jnp.int32(0), jnp.int32(1)))

        acc = merged[pl.ds(grp * L, L)]
        rev = lax.rev(acc, (0,))
        c = plsc.cumsum(rev)
        mge = c >= rg
        i0 = _scal(plsc.all_reduce_ffs(mge))
        msel = lanes == i0
        ci0 = jnp.sum(jnp.where(msel, c, zero16))
        ri0 = jnp.sum(jnp.where(msel, rev, zero16))
        bucket = grp * L + (15 - i0)
        r_next = rg - (ci0 - ri0)
        return bucket, r_next

    def histo(src, n_src, shift):
        clear_hist()
        nv = (n_src + L - 1) // L
        def body(i, _):
            sv = src[pl.ds(i * L, L)]
            inb = (i * L + lanes) < n_src
            bkt = ((sv >> shift) & 0xFF) + laneoff
            plsc.addupdate_scatter(hist, [bkt], ones, mask=inb)
            return 0
        lax.fori_loop(0, nv, body, 0)

    def compact(src, dst, n_src, shift, p):
        nv = (n_src + L - 1) // L
        def body(i, off):
            sv = src[pl.ds(i * L, L)]
            inb = (i * L + lanes) < n_src
            m = jnp.logical_and((sv >> shift) == p, inb)
            mi = jnp.where(m, ones, zero16)
            pos = off + plsc.cumsum(mi) - 1
            plsc.store_scatter(dst, [pos], sv, mask=m)
            return off + plsc.all_reduce_population_count(m)
        off = lax.fori_loop(0, nv, body, zero16)
        return jnp.max(off)

    def row_body(rr, _):
        row = wid * ROWS_PER_W + rr
        pltpu.sync_copy(x_hbm.at[row], xv)

        clear_hist()
        def p1_body(i, _):
            xf = xv[pl.ds(i * L, L)]
            b = lax.bitcast_convert_type(xf, jnp.int32)
            s = jnp.where(b < 0, b ^ jnp.int32(0x7FFFFFFF), b)
            sa[pl.ds(i * L, L)] = s
            bkt = (s >> 24) + 128 + laneoff
            plsc.addupdate_scatter(hist, [bkt], ones, mask=true16)
            return 0
        lax.fori_loop(0, NV, p1_body, 0)

        b1, r = search(jnp.int32(K))
        p1 = b1 - 128
        n1 = compact(sa, cb, jnp.int32(N), 24, p1)

        histo(cb, n1, 16)
        b2, r = search(r)
        p2 = (p1 << 8) | b2
        n2 = compact(cb, sa, n1, 16, p2)

        histo(sa, n2, 8)
        b3, r = search(r)
        p3 = (p2 << 8) | b3
        n3 = compact(sa, cb, n2, 8, p3)

        histo(cb, n3, 0)
        b4, r = search(r)
        thr = (p3 << 8) | b4

        def ob(i, _):
            xf = xv[pl.ds(i * L, L)]
            b = lax.bitcast_convert_type(xf, jnp.int32)
            s = jnp.where(b < 0, b ^ jnp.int32(0x7FFFFFFF), b)
            xv[pl.ds(i * L, L)] = jnp.where(s >= thr, xf, fzero16)
            return 0
        lax.fori_loop(0, NV, ob, 0)

        pltpu.sync_copy(xv, out_hbm.at[row])
        return 0

    lax.fori_loop(0, ROWS_PER_W, row_body, 0)


def kernel(x):
    return _topk_mask(x)


# speedup vs baseline: 9.9122x; 1.2854x over previous
import functools
import math

import jax
import jax.numpy as jnp
from jax import lax
from jax.experimental import pallas as pl
from jax.experimental.pallas import tpu as pltpu
from jax.experimental.pallas import tpu_sc as plsc

B = 128
N = 32768
K = math.ceil(0.05 * N)

NC = 2
NS = 16
L = 16
NW = NC * NS
ROWS_PER_W = B // NW
NV = N // L
NBKT = 256
HSTRIDE = 257
HWORDS = 264 * L
U = 8


def _scal(v):
    if getattr(v, "ndim", 0) == 1:
        return jnp.max(v)
    return v


_mesh = plsc.VectorSubcoreMesh(core_axis_name="c", subcore_axis_name="s")


@functools.partial(
    pl.kernel,
    mesh=_mesh,
    out_type=jax.ShapeDtypeStruct((B, N), jnp.float32),
    compiler_params=pltpu.CompilerParams(needs_layout_passes=False),
    scratch_types=[
        pltpu.VMEM((N,), jnp.float32),
        pltpu.VMEM((N,), jnp.int32),
        pltpu.VMEM((N,), jnp.int32),
        pltpu.VMEM((HWORDS,), jnp.int32),
        pltpu.VMEM((NBKT,), jnp.int32),
    ],
)
def _topk_mask(x_hbm, out_hbm, xv, sa, cb, hist, merged):
    wid = lax.axis_index("s") * NC + lax.axis_index("c")
    lanes = lax.iota(jnp.int32, L)
    laneoff = lanes * HSTRIDE
    ones = jnp.ones((L,), jnp.int32)
    zero16 = jnp.zeros((L,), jnp.int32)
    fzero16 = jnp.zeros((L,), jnp.float32)
    true16 = lanes >= 0

    def clear_hist():
        def body(i, _):
            base = i * (L * U)
            for u in range(U):
                hist[pl.ds(base + u * L, L)] = zero16
            return 0
        lax.fori_loop(0, HWORDS // (L * U), body, 0)

    def search(r0):
        def ga_body(t, carry):
            r, found, grp, rg = carry
            g = 15 - t
            acc = zero16
            for l in range(L):
                acc = acc + hist[pl.ds(l * HSTRIDE + g * L, L)]
            merged[pl.ds(g * L, L)] = acc
            tot = jnp.sum(acc)
            hit = jnp.logical_and(found == 0, tot >= r)
            grp = jnp.where(hit, g, grp)
            rg = jnp.where(hit, r, rg)
            r = jnp.where(jnp.logical_and(found == 0, jnp.logical_not(hit)),
                          r - tot, r)
            found = jnp.where(hit, jnp.int32(1), found)
            return (r, found, grp, rg)

        r, _found, grp, rg = lax.fori_loop(
            0, 16, ga_body,
            (r0, jnp.int32(0), jnp.int32(0), jnp.int32(1)))

        acc = merged[pl.ds(grp * L, L)]
        rev = lax.rev(acc, (0,))
        c = plsc.cumsum(rev)
        mge = c >= rg
        i0 = _scal(plsc.all_reduce_ffs(mge))
        msel = lanes == i0
        ci0 = jnp.sum(jnp.where(msel, c, zero16))
        ri0 = jnp.sum(jnp.where(msel, rev, zero16))
        bucket = grp * L + (15 - i0)
        r_next = rg - (ci0 - ri0)
        return bucket, r_next

    def histo(src, n_src, shift):
        clear_hist()
        nv = (n_src + L * U - 1) // (L * U)
        def body(i, _):
            base = i * (L * U)
            for u in range(U):
                sv = src[pl.ds(base + u * L, L)]
                inb = (base + u * L + lanes) < n_src
                bkt = ((sv >> shift) & 0xFF) + laneoff
                plsc.addupdate_scatter(hist, [bkt], ones, mask=inb)
            return 0
        lax.fori_loop(0, nv, body, 0)

    def compact(src, dst, n_src, shift, p):
        nv = (n_src + L * U - 1) // (L * U)
        def body(i, off):
            base = i * (L * U)
            for u in range(U):
                sv = src[pl.ds(base + u * L, L)]
                inb = (base + u * L + lanes) < n_src
                m = jnp.logical_and((sv >> shift) == p, inb)
                mi = jnp.where(m, ones, zero16)
                pos = off + plsc.cumsum(mi) - 1
                plsc.store_scatter(dst, [pos], sv, mask=m)
                off = off + plsc.all_reduce_population_count(m)
            return off
        off = lax.fori_loop(0, nv, body, zero16)
        return jnp.max(off)

    def row_body(rr, _):
        row = wid * ROWS_PER_W + rr
        pltpu.sync_copy(x_hbm.at[row], xv)

        clear_hist()
        def p1_body(i, _):
            base = i * (L * U)
            for u in range(U):
                xf = xv[pl.ds(base + u * L, L)]
                b = lax.bitcast_convert_type(xf, jnp.int32)
                s = jnp.where(b < 0, b ^ jnp.int32(0x7FFFFFFF), b)
                sa[pl.ds(base + u * L, L)] = s
                bkt = (s >> 24) + 128 + laneoff
                plsc.addupdate_scatter(hist, [bkt], ones, mask=true16)
            return 0
        lax.fori_loop(0, NV // U, p1_body, 0)

        b1, r = search(jnp.int32(K))
        p1 = b1 - 128
        n1 = compact(sa, cb, jnp.int32(N), 24, p1)

        histo(cb, n1, 16)
        b2, r = search(r)
        p2 = (p1 << 8) | b2
        n2 = compact(cb, sa, n1, 16, p2)

        histo(sa, n2, 8)
        b3, r = search(r)
        p3 = (p2 << 8) | b3
        n3 = compact(sa, cb, n2, 8, p3)

        histo(cb, n3, 0)
        b4, r = search(r)
        thr = (p3 << 8) | b4

        def ob(i, _):
            base = i * (L * U)
            for u in range(U):
                xf = xv[pl.ds(base + u * L, L)]
                b = lax.bitcast_convert_type(xf, jnp.int32)
                s = jnp.where(b < 0, b ^ jnp.int32(0x7FFFFFFF), b)
                xv[pl.ds(base + u * L, L)] = jnp.where(s >= thr, xf, fzero16)
            return 0
        lax.fori_loop(0, NV // U, ob, 0)

        pltpu.sync_copy(xv, out_hbm.at[row])
        return 0

    lax.fori_loop(0, ROWS_PER_W, row_body, 0)


def kernel(x):
    return _topk_mask(x)
